# Initial kernel scaffold; baseline (speedup 1.0000x reference)
import jax
import jax.numpy as jnp
from jax.experimental import pallas as pl


def kernel(input, target):
    raise NotImplementedError("write your pallas kernel here")


# speedup vs baseline: 5.5929x; 5.5929x over previous
import math

import jax
import jax.numpy as jnp
from jax.experimental import pallas as pl

_N = 64
_C = 8
_DIRS = [[i, j, k] for i in (-1, 0, 1) for j in (-1, 0, 1) for k in (-1, 0, 1)]
_DIRS.remove([0, 0, 0])
_LN2 = math.log(2.0)


def _shift1(x, axis, d):
    if d == 0:
        return x
    n = x.shape[axis]
    pad = jnp.zeros_like(jax.lax.slice_in_dim(x, 0, 1, axis=axis))
    if d == 1:
        body = jax.lax.slice_in_dim(x, 1, n, axis=axis)
        return jnp.concatenate([body, pad], axis=axis)
    body = jax.lax.slice_in_dim(x, 0, n - 1, axis=axis)
    return jnp.concatenate([pad, body], axis=axis)


def _shift3(x, d):
    for ax in range(3):
        x = _shift1(x, ax, d[ax])
    return x


def _prep_kernel(p_ref, t_ref, kl_ref, e_ref, f0_ref):
    pcs = [p_ref[c] for c in range(_C)]
    logs = [pc * jnp.log(jnp.where(pc > 0, pc, 1.0)) for pc in pcs]
    e = sum(logs) * (1.0 / _C)

    kls = []
    for ax in (2, 1, 0):
        dot = sum(pcs[c] * _shift1(pcs[c], ax, 1) for c in range(_C)) * (1.0 / _C)
        kl = _shift1(e, ax, 1) - dot
        idx = jax.lax.broadcasted_iota(jnp.int32, kl.shape, ax)
        kls.append(jnp.where(idx == _N - 1, -1.0, kl))
    kl_ref[...] = jnp.maximum(jnp.maximum(kls[0], kls[1]), kls[2])
    e_ref[...] = e

    t = t_ref[...].astype(jnp.float32)
    s = _shift1(t, 2, 1) + _shift1(t, 1, 1) + _shift1(t, 0, 1)
    gdb = (t * 3.0) != s
    f0_ref[...] = jnp.where(gdb, 0.0, 1e12)


def _dt_kernel(f_ref, o_ref):
    f = f_ref[...]
    i = jax.lax.broadcasted_iota(jnp.int32, (_N, _N), 0)
    j = jax.lax.broadcasted_iota(jnp.int32, (_N, _N), 1)
    cost = ((i - j).astype(jnp.float32)) ** 2
    o_ref[...] = jnp.min(f[:, None, :] + cost[None, :, :], axis=-1)


def _loss_kernel(p_ref, e_ref, fsq_ref, kl_ref, thr_ref, sum_ref, cnt_ref):
    pcs = [p_ref[c] for c in range(_C)]
    e = e_ref[...]
    gd = jnp.sqrt(fsq_ref[...])
    d_iota = jax.lax.broadcasted_iota(jnp.int32, e.shape, 0)
    h_iota = jax.lax.broadcasted_iota(jnp.int32, e.shape, 1)

    def klr_for(d):
        dot = sum(pcs[c] * _shift3(pcs[c], d) for c in range(_C)) * (1.0 / _C)
        k = jnp.exp(_shift3(e, d) - dot)
        if d[1] != 0:
            k = jnp.where(d_iota == (_N - 1 if d[1] == 1 else 0), k, 0.0)
        if d[2] != 0:
            k = jnp.where(h_iota == (_N - 1 if d[2] == 1 else 0), k, 0.0)
        return k

    zero = jnp.zeros_like(e)
    S = zero
    min_d = jnp.full_like(e, jnp.inf)
    x_min = zero
    for d in _DIRS:
        distk = _shift3(gd, d)
        klr = zero if d[0] == 1 else klr_for(d)
        S = S + klr
        upd = distk < min_d
        x_min = jnp.where(upd, klr, x_min)
        min_d = jnp.where(upd, distk, min_d)

    sum_f = jnp.full_like(e, 9.0 * _LN2)
    for d in _DIRS:
        if d[0] == 1:
            continue
        x = klr_for(d) / S
        sum_f = sum_f + jnp.maximum(x, 0.0) + jnp.log1p(jnp.exp(-jnp.abs(x)))

    a = 0.2 / 26.0
    meanbce = (sum_f - (a + (0.8 - a) * (x_min / S))) * (1.0 / 26.0)
    weight = jnp.minimum(gd, 20.0) / 20.0
    loss = weight * meanbce
    loss = jnp.where(gd != 0.0, loss, 0.0)
    m = kl_ref[...] >= thr_ref[0, 0]
    loss = jnp.where(m, loss, 0.0)
    sum_ref[...] = jnp.reshape(jnp.sum(loss), (1, 1))
    cnt_ref[...] = jnp.reshape(jnp.sum(m.astype(jnp.float32)), (1, 1))


def _dt_pass(f2d):
    return pl.pallas_call(
        _dt_kernel,
        out_shape=jax.ShapeDtypeStruct((_N * _N, _N), jnp.float32),
        grid=(8,),
        in_specs=[pl.BlockSpec((_N * _N // 8, _N), lambda i: (i, 0))],
        out_specs=pl.BlockSpec((_N * _N // 8, _N), lambda i: (i, 0)),
    )(f2d)


def kernel(input, target):
    p = input[0]
    t = target[0]
    vol = jax.ShapeDtypeStruct((_N, _N, _N), jnp.float32)
    kl_vals, e, f0 = pl.pallas_call(
        _prep_kernel, out_shape=[vol, vol, vol]
    )(p, t)

    thr = jnp.quantile(kl_vals, 0.99).astype(jnp.float32)

    f = f0
    y = _dt_pass(f.transpose(1, 2, 0).reshape(_N * _N, _N))
    f = y.reshape(_N, _N, _N).transpose(2, 0, 1)
    y = _dt_pass(f.transpose(0, 2, 1).reshape(_N * _N, _N))
    f = y.reshape(_N, _N, _N).transpose(0, 2, 1)
    y = _dt_pass(f.reshape(_N * _N, _N))
    fsq = y.reshape(_N, _N, _N)

    s11 = jax.ShapeDtypeStruct((1, 1), jnp.float32)
    lsum, cnt = pl.pallas_call(
        _loss_kernel,
        out_shape=[s11, s11],
    )(p, e, fsq, kl_vals, thr.reshape(1, 1))
    return lsum[0, 0] / cnt[0, 0]
